# baseline (speedup 1.0000x reference)
import functools

import jax
import jax.numpy as jnp
from jax import lax
from jax.experimental import pallas as pl
from jax.experimental.pallas import tpu as pltpu
from jax.experimental.pallas import tpu_sc as plsc

N = 10000
D = 128
E = 320000

NC = 2
NS = 16
NW = NC * NS
CH = 128
BK = 40
NB0 = 2
NB1 = 2
CPW0 = BK * NB0
CPW1 = BK * NB1
TCH = NS * (CPW0 + CPW1)
EP = TCH * CH
NP = 10240
RPT = NP // NS

_mesh = functools.partial(
    plsc.VectorSubcoreMesh,
    core_axis_name="c", subcore_axis_name="s", num_cores=NC, num_subcores=NS,
)


def _chunk_base(c, s):
    return lax.select(c == 0, s * CPW0, NS * CPW0 + s * CPW1)


def _deg_body(colp, out, colv, degv):
    c = lax.axis_index("c")
    s = lax.axis_index("s")
    wid = c * NS + s
    base = _chunk_base(c, s)

    def z(i, _):
        degv[pl.ds(i * 16, 16)] = jnp.zeros((16,), jnp.float32)
        return 0
    lax.fori_loop(0, NP // 16, z, 0)

    ones16 = jnp.full((16,), 1.0, jnp.float32)

    for blk in range(NB0):
        @pl.when((c == 0) | (blk < NB1))
        def _():
            pltpu.sync_copy(colp.at[pl.ds(base + blk * BK, BK)], colv)

            def body(j, _):
                for k in range(CH // 16):
                    idx = colv[j, pl.ds(k * 16, 16)]
                    plsc.addupdate_scatter(degv, [idx], ones16)
                return 0
            lax.fori_loop(0, BK, body, 0)

    pltpu.sync_copy(degv, out.at[wid])


def _sc_degree(colp):
    return pl.kernel(
        _deg_body,
        out_type=jax.ShapeDtypeStruct((NW, NP), jnp.float32),
        mesh=_mesh(),
        scratch_types=[
            pltpu.VMEM((BK, CH), jnp.int32),
            pltpu.VMEM((NP,), jnp.float32),
        ],
        compiler_params=pltpu.CompilerParams(needs_layout_passes=False),
    )(colp)


def _agg_body(g, rowp, colp, out, rowb, colb, buf0, buf1, accsp,
              sem0, sem1, sem2, sem3):
    c = lax.axis_index("c")
    s = lax.axis_index("s")
    base = _chunk_base(c, s)

    def fill(i, _):
        for k in range(8):
            buf0[i, pl.ds(k * 16, 16)] = jnp.zeros((16,), jnp.float32)
        return 0
    lax.fori_loop(0, CH, fill, 0)
    for r in range(RPT // CH):
        pltpu.sync_copy(buf0, accsp.at[pl.ds(s * RPT + r * CH, CH)])
    plsc.subcore_barrier()

    for blk in range(NB0):
        @pl.when((c == 0) | (blk < NB1))
        def _():
            b0 = base + blk * BK
            ia = pltpu.async_copy(rowp.at[pl.ds(b0, BK)], rowb, sem0)
            ib = pltpu.async_copy(colp.at[pl.ds(b0, BK)], colb, sem1)
            ia.wait()
            ib.wait()

            pltpu.async_copy(g.at[rowb.at[0]], buf0, sem0)
            pltpu.async_copy(g.at[rowb.at[1]], buf1, sem1)

            def body(i, _):
                j0 = 2 * i
                pltpu.make_async_copy(g.at[rowb.at[j0]], buf0, sem0).wait()
                pltpu.async_copy(buf0, accsp.at[colb.at[j0]], sem2, add=True)
                pltpu.make_async_copy(g.at[rowb.at[j0 + 1]], buf1, sem1).wait()
                pltpu.async_copy(buf1, accsp.at[colb.at[j0 + 1]], sem3, add=True)
                pltpu.make_async_copy(buf0, accsp.at[colb.at[j0]], sem2).wait()
                pltpu.async_copy(g.at[rowb.at[j0 + 2]], buf0, sem0)
                pltpu.make_async_copy(buf1, accsp.at[colb.at[j0 + 1]], sem3).wait()
                pltpu.async_copy(g.at[rowb.at[j0 + 3]], buf1, sem1)
                return 0
            lax.fori_loop(0, BK // 2 - 1, body, 0)

            pltpu.make_async_copy(g.at[rowb.at[BK - 2]], buf0, sem0).wait()
            pltpu.sync_copy(buf0, accsp.at[colb.at[BK - 2]], add=True)
            pltpu.make_async_copy(g.at[rowb.at[BK - 1]], buf1, sem1).wait()
            pltpu.sync_copy(buf1, accsp.at[colb.at[BK - 1]], add=True)

    plsc.subcore_barrier()
    pltpu.sync_copy(accsp.at[pl.ds(s * RPT, RPT)], out.at[c, pl.ds(s * RPT, RPT)])


def _sc_aggregate(g, rowp, colp):
    return pl.kernel(
        _agg_body,
        out_type=jax.ShapeDtypeStruct((NC, NP, D), jnp.float32),
        mesh=_mesh(),
        scratch_types=[
            pltpu.VMEM((BK, CH), jnp.int32),
            pltpu.VMEM((BK, CH), jnp.int32),
            pltpu.VMEM((CH, D), jnp.float32),
            pltpu.VMEM((CH, D), jnp.float32),
            pltpu.VMEM_SHARED((NP, D), jnp.float32),
            pltpu.SemaphoreType.DMA,
            pltpu.SemaphoreType.DMA,
            pltpu.SemaphoreType.DMA,
            pltpu.SemaphoreType.DMA,
        ],
    )(g, rowp, colp)


_R = 1024


def _tc1_body(x_ref, w_ref, deg_ref, h_ref, g_ref, dis_ref):
    d = deg_ref[...]
    deg = 1.0 + jnp.sum(d, axis=0, keepdims=True)
    dis = jnp.transpose(lax.rsqrt(deg))
    h = jnp.dot(x_ref[...], w_ref[...], preferred_element_type=jnp.float32)
    h_ref[...] = h
    g_ref[...] = dis * h
    dis_ref[...] = jnp.broadcast_to(dis, (_R, D))


def _tc_prep(xp, W1, degp):
    return pl.pallas_call(
        _tc1_body,
        grid=(NP // _R,),
        in_specs=[
            pl.BlockSpec((_R, D), lambda i: (i, 0)),
            pl.BlockSpec((D, D), lambda i: (0, 0)),
            pl.BlockSpec((NW, _R), lambda i: (0, i)),
        ],
        out_specs=[
            pl.BlockSpec((_R, D), lambda i: (i, 0)),
            pl.BlockSpec((_R, D), lambda i: (i, 0)),
            pl.BlockSpec((_R, D), lambda i: (i, 0)),
        ],
        out_shape=[
            jax.ShapeDtypeStruct((NP, D), jnp.float32),
            jax.ShapeDtypeStruct((NP, D), jnp.float32),
            jax.ShapeDtypeStruct((NP, D), jnp.float32),
        ],
    )(xp, W1, degp)


def _tc2_body(acc_ref, h_ref, dis_ref, b_ref, w_ref, h2_ref, g2_ref):
    a = acc_ref[0] + acc_ref[1]
    dis = dis_ref[...]
    pre = dis * a + dis * dis * h_ref[...] + b_ref[...]
    r = jnp.maximum(pre, 0.0)
    h2 = jnp.dot(r, w_ref[...], preferred_element_type=jnp.float32)
    h2_ref[...] = h2
    g2_ref[...] = dis * h2


def _tc_mid(acc1, h1, disb, b1, W2):
    return pl.pallas_call(
        _tc2_body,
        grid=(NP // _R,),
        in_specs=[
            pl.BlockSpec((NC, _R, D), lambda i: (0, i, 0)),
            pl.BlockSpec((_R, D), lambda i: (i, 0)),
            pl.BlockSpec((_R, D), lambda i: (i, 0)),
            pl.BlockSpec((1, D), lambda i: (0, 0)),
            pl.BlockSpec((D, D), lambda i: (0, 0)),
        ],
        out_specs=[
            pl.BlockSpec((_R, D), lambda i: (i, 0)),
            pl.BlockSpec((_R, D), lambda i: (i, 0)),
        ],
        out_shape=[
            jax.ShapeDtypeStruct((NP, D), jnp.float32),
            jax.ShapeDtypeStruct((NP, D), jnp.float32),
        ],
    )(acc1, h1, disb, b1, W2)


def _tc3_body(acc_ref, h_ref, dis_ref, b_ref, out_ref):
    a = acc_ref[0] + acc_ref[1]
    dis = dis_ref[...]
    out_ref[...] = dis * a + dis * dis * h_ref[...] + b_ref[...]


def _tc_final(acc2, h2, disb, b2):
    return pl.pallas_call(
        _tc3_body,
        grid=(NP // _R,),
        in_specs=[
            pl.BlockSpec((NC, _R, D), lambda i: (0, i, 0)),
            pl.BlockSpec((_R, D), lambda i: (i, 0)),
            pl.BlockSpec((_R, D), lambda i: (i, 0)),
            pl.BlockSpec((1, D), lambda i: (0, 0)),
        ],
        out_specs=pl.BlockSpec((_R, D), lambda i: (i, 0)),
        out_shape=jax.ShapeDtypeStruct((NP, D), jnp.float32),
    )(acc2, h2, disb, b2)


def kernel(x, edge_index, W1, b1, W2, b2):
    row = edge_index[0]
    col = edge_index[1]
    spread = N + (jnp.arange(EP - E, dtype=jnp.int32) % (NP - N))
    rowp = jnp.concatenate([row, spread])
    colp = jnp.concatenate([col, spread])
    rowp = rowp.reshape(TCH, CH)
    colp = colp.reshape(TCH, CH)
    xp = jnp.pad(x, ((0, NP - N), (0, 0)))
    b1r = b1.reshape(1, D)
    b2r = b2.reshape(1, D)

    degp = _sc_degree(colp)
    h1, g1, disb = _tc_prep(xp, W1, degp)
    acc1 = _sc_aggregate(g1, rowp, colp)
    h2, g2 = _tc_mid(acc1, h1, disb, b1r, W2)
    acc2 = _sc_aggregate(g2, rowp, colp)
    out = _tc_final(acc2, h2, disb, b2r)
    return out[:N]


# speedup vs baseline: 1.2807x; 1.2807x over previous
import jax
import jax.numpy as jnp
from jax import lax
from jax.experimental import pallas as pl
from jax.experimental.pallas import tpu as pltpu
from jax.experimental.pallas import tpu_sc as plsc

N = 10000
D = 128
E = 320000

NC = 2
NS = 16
NW = NC * NS
CH = 128
TCH = 2504
EP = TCH * CH
B0 = 40
CPT = 2 * B0
LAST = TCH - (NW - 1) * CPT
NPA = 10240
RPT = NPA // NS

_MESH = dict(core_axis_name="c", subcore_axis_name="s",
             num_cores=NC, num_subcores=NS)


def _wid_base(c, s):
    wid = c * NS + s
    return wid, wid * CPT


def _deg_body(er, out, colv, degv):
    c = lax.axis_index("c")
    s = lax.axis_index("s")
    wid, base = _wid_base(c, s)

    def z(i, _):
        degv[pl.ds(i * 16, 16)] = jnp.zeros((16,), jnp.float32)
        return 0
    lax.fori_loop(0, NPA // 16, z, 0)

    ones16 = jnp.full((16,), 1.0, jnp.float32)

    def count_block(b0, nchunks):
        pltpu.sync_copy(er.at[1, pl.ds(b0, nchunks)], colv.at[pl.ds(0, nchunks)])

        def body(j, _):
            for k in range(CH // 16):
                idx = colv[j, pl.ds(k * 16, 16)]
                plsc.addupdate_scatter(degv, [idx], ones16)
            return 0
        lax.fori_loop(0, nchunks, body, 0)

    @pl.when(wid < NW - 1)
    def _():
        count_block(base, B0)
        count_block(base + B0, B0)

    @pl.when(wid == NW - 1)
    def _():
        count_block(base, LAST)

    pltpu.sync_copy(degv, out.at[wid])


def _sc_degree(er):
    return pl.kernel(
        _deg_body,
        out_type=jax.ShapeDtypeStruct((NW, NPA), jnp.float32),
        mesh=plsc.VectorSubcoreMesh(**_MESH),
        scratch_types=[
            pltpu.VMEM((B0, CH), jnp.int32),
            pltpu.VMEM((NPA,), jnp.float32),
        ],
        compiler_params=pltpu.CompilerParams(needs_layout_passes=False),
    )(er)


def _agg_body(g, er, out, rowb, colb, buf0, buf1, accsp, sem0, sem1):
    c = lax.axis_index("c")
    s = lax.axis_index("s")
    wid, base = _wid_base(c, s)

    def fill(i, _):
        for k in range(8):
            buf0[i, pl.ds(k * 16, 16)] = jnp.zeros((16,), jnp.float32)
        return 0
    lax.fori_loop(0, CH, fill, 0)
    for r in range(RPT // CH):
        pltpu.sync_copy(buf0, accsp.at[pl.ds(s * RPT + r * CH, CH)])
    plsc.subcore_barrier()

    def run_block(b0, nchunks):
        ia = pltpu.async_copy(er.at[0, pl.ds(b0, nchunks)],
                              rowb.at[pl.ds(0, nchunks)], sem0)
        ib = pltpu.async_copy(er.at[1, pl.ds(b0, nchunks)],
                              colb.at[pl.ds(0, nchunks)], sem1)
        ia.wait()
        ib.wait()

        pltpu.async_copy(g.at[rowb.at[0]], buf0, sem0)
        pltpu.async_copy(g.at[rowb.at[1]], buf1, sem1)

        def body(i, _):
            j0 = 2 * i
            pltpu.make_async_copy(g.at[rowb.at[j0]], buf0, sem0).wait()
            pltpu.sync_copy(buf0, accsp.at[colb.at[j0]], add=True)
            pltpu.async_copy(g.at[rowb.at[j0 + 2]], buf0, sem0)
            pltpu.make_async_copy(g.at[rowb.at[j0 + 1]], buf1, sem1).wait()
            pltpu.sync_copy(buf1, accsp.at[colb.at[j0 + 1]], add=True)
            pltpu.async_copy(g.at[rowb.at[j0 + 3]], buf1, sem1)
            return 0
        lax.fori_loop(0, nchunks // 2 - 1, body, 0)

        pltpu.make_async_copy(g.at[rowb.at[nchunks - 2]], buf0, sem0).wait()
        pltpu.sync_copy(buf0, accsp.at[colb.at[nchunks - 2]], add=True)
        pltpu.make_async_copy(g.at[rowb.at[nchunks - 1]], buf1, sem1).wait()
        pltpu.sync_copy(buf1, accsp.at[colb.at[nchunks - 1]], add=True)

    @pl.when(wid < NW - 1)
    def _():
        run_block(base, B0)
        run_block(base + B0, B0)

    @pl.when(wid == NW - 1)
    def _():
        run_block(base, LAST)

    plsc.subcore_barrier()
    pltpu.sync_copy(accsp.at[pl.ds(s * RPT, RPT)], out.at[c, pl.ds(s * RPT, RPT)])


def _sc_aggregate(g, er):
    return pl.kernel(
        _agg_body,
        out_type=jax.ShapeDtypeStruct((NC, NPA, D), jnp.float32),
        mesh=plsc.VectorSubcoreMesh(**_MESH),
        scratch_types=[
            pltpu.VMEM((B0, CH), jnp.int32),
            pltpu.VMEM((B0, CH), jnp.int32),
            pltpu.VMEM((CH, D), jnp.float32),
            pltpu.VMEM((CH, D), jnp.float32),
            pltpu.VMEM_SHARED((NPA, D), jnp.float32),
            pltpu.SemaphoreType.DMA,
            pltpu.SemaphoreType.DMA,
        ],
    )(g, er)


_R = 1000


def _tc1_body(x_ref, w_ref, deg_ref, h_ref, g_ref, dis_ref):
    d = deg_ref[...]
    deg = 1.0 + jnp.sum(d, axis=1, keepdims=True)
    dis = lax.rsqrt(deg)
    h = jnp.dot(x_ref[...], w_ref[...], preferred_element_type=jnp.float32)
    h_ref[...] = h
    g_ref[...] = dis * h
    dis_ref[...] = jnp.broadcast_to(dis, (_R, D))


def _tc_prep(x, W1, degp):
    return pl.pallas_call(
        _tc1_body,
        grid=(N // _R,),
        in_specs=[
            pl.BlockSpec((_R, D), lambda i: (i, 0)),
            pl.BlockSpec((D, D), lambda i: (0, 0)),
            pl.BlockSpec((_R, NW), lambda i: (i, 0)),
        ],
        out_specs=[
            pl.BlockSpec((_R, D), lambda i: (i, 0)),
            pl.BlockSpec((_R, D), lambda i: (i, 0)),
            pl.BlockSpec((_R, D), lambda i: (i, 0)),
        ],
        out_shape=[
            jax.ShapeDtypeStruct((N, D), jnp.float32),
            jax.ShapeDtypeStruct((N, D), jnp.float32),
            jax.ShapeDtypeStruct((N, D), jnp.float32),
        ],
    )(x, W1, degp)


def _tc2_body(acc_ref, h_ref, dis_ref, b_ref, w_ref, h2_ref, g2_ref):
    a = acc_ref[0] + acc_ref[1]
    dis = dis_ref[...]
    pre = dis * a + dis * dis * h_ref[...] + b_ref[...]
    r = jnp.maximum(pre, 0.0)
    h2 = jnp.dot(r, w_ref[...], preferred_element_type=jnp.float32)
    h2_ref[...] = h2
    g2_ref[...] = dis * h2


def _tc_mid(acc1, h1, disb, b1, W2):
    return pl.pallas_call(
        _tc2_body,
        grid=(N // _R,),
        in_specs=[
            pl.BlockSpec((NC, _R, D), lambda i: (0, i, 0)),
            pl.BlockSpec((_R, D), lambda i: (i, 0)),
            pl.BlockSpec((_R, D), lambda i: (i, 0)),
            pl.BlockSpec((1, D), lambda i: (0, 0)),
            pl.BlockSpec((D, D), lambda i: (0, 0)),
        ],
        out_specs=[
            pl.BlockSpec((_R, D), lambda i: (i, 0)),
            pl.BlockSpec((_R, D), lambda i: (i, 0)),
        ],
        out_shape=[
            jax.ShapeDtypeStruct((N, D), jnp.float32),
            jax.ShapeDtypeStruct((N, D), jnp.float32),
        ],
    )(acc1, h1, disb, b1, W2)


def _tc3_body(acc_ref, h_ref, dis_ref, b_ref, out_ref):
    a = acc_ref[0] + acc_ref[1]
    dis = dis_ref[...]
    out_ref[...] = dis * a + dis * dis * h_ref[...] + b_ref[...]


def _tc_final(acc2, h2, disb, b2):
    return pl.pallas_call(
        _tc3_body,
        grid=(N // _R,),
        in_specs=[
            pl.BlockSpec((NC, _R, D), lambda i: (0, i, 0)),
            pl.BlockSpec((_R, D), lambda i: (i, 0)),
            pl.BlockSpec((_R, D), lambda i: (i, 0)),
            pl.BlockSpec((1, D), lambda i: (0, 0)),
        ],
        out_specs=pl.BlockSpec((_R, D), lambda i: (i, 0)),
        out_shape=jax.ShapeDtypeStruct((N, D), jnp.float32),
    )(acc2, h2, disb, b2)


def kernel(x, edge_index, W1, b1, W2, b2):
    k = jnp.arange(EP - E, dtype=jnp.int32)
    pad = jnp.stack([k % N, N + k % (NPA - N)])
    er = jnp.concatenate([edge_index, pad], axis=1).reshape(2, TCH, CH)
    b1r = b1.reshape(1, D)
    b2r = b2.reshape(1, D)

    degp = jnp.transpose(_sc_degree(er))
    h1, g1, disb = _tc_prep(x, W1, degp)
    acc1 = _sc_aggregate(g1, er)
    h2, g2 = _tc_mid(acc1, h1, disb, b1r, W2)
    acc2 = _sc_aggregate(g2, er)
    return _tc_final(acc2, h2, disb, b2r)


# speedup vs baseline: 1.2838x; 1.0024x over previous
import jax
import jax.numpy as jnp
from jax import lax
from jax.experimental import pallas as pl
from jax.experimental.pallas import tpu as pltpu
from jax.experimental.pallas import tpu_sc as plsc

N = 10000
D = 128
E = 320000

NC = 2
NS = 16
NW = NC * NS
CH = 128
TCH = 2504
EP = TCH * CH
B0 = 40
CPT = 2 * B0
LAST = TCH - (NW - 1) * CPT
NPA = 10240
RPT = NPA // NS

_MESH = dict(core_axis_name="c", subcore_axis_name="s",
             num_cores=NC, num_subcores=NS)


def _wid_base(c, s):
    wid = c * NS + s
    return wid, wid * CPT


def _deg_body(er, out, colv, degv):
    c = lax.axis_index("c")
    s = lax.axis_index("s")
    wid, base = _wid_base(c, s)

    def z(i, _):
        degv[pl.ds(i * 16, 16)] = jnp.zeros((16,), jnp.float32)
        return 0
    lax.fori_loop(0, NPA // 16, z, 0)

    ones16 = jnp.full((16,), 1.0, jnp.float32)

    def count_block(b0, nchunks):
        pltpu.sync_copy(er.at[1, pl.ds(b0, nchunks)], colv.at[pl.ds(0, nchunks)])

        def body(j, _):
            for k in range(CH // 16):
                idx = colv[j, pl.ds(k * 16, 16)]
                plsc.addupdate_scatter(degv, [idx], ones16)
            return 0
        lax.fori_loop(0, nchunks, body, 0)

    @pl.when(wid < NW - 1)
    def _():
        count_block(base, B0)
        count_block(base + B0, B0)

    @pl.when(wid == NW - 1)
    def _():
        count_block(base, LAST)

    pltpu.sync_copy(degv, out.at[wid])


def _sc_degree(er):
    return pl.kernel(
        _deg_body,
        out_type=jax.ShapeDtypeStruct((NW, NPA), jnp.float32),
        mesh=plsc.VectorSubcoreMesh(**_MESH),
        scratch_types=[
            pltpu.VMEM((B0, CH), jnp.int32),
            pltpu.VMEM((NPA,), jnp.float32),
        ],
        compiler_params=pltpu.CompilerParams(needs_layout_passes=False),
    )(er)


def _agg_body(g, er, out, rowb, colb, buf0, buf1, accsp, sem0, sem1):
    c = lax.axis_index("c")
    s = lax.axis_index("s")
    wid, base = _wid_base(c, s)

    def fill(i, _):
        for k in range(8):
            buf0[i, pl.ds(k * 16, 16)] = jnp.zeros((16,), jnp.float32)
        return 0
    lax.fori_loop(0, CH, fill, 0)
    for r in range(RPT // CH):
        pltpu.sync_copy(buf0, accsp.at[pl.ds(s * RPT + r * CH, CH)])
    plsc.subcore_barrier()

    def run_block(b0, nchunks):
        ia = pltpu.async_copy(er.at[0, pl.ds(b0, nchunks)],
                              rowb.at[pl.ds(0, nchunks)], sem0)
        ib = pltpu.async_copy(er.at[1, pl.ds(b0, nchunks)],
                              colb.at[pl.ds(0, nchunks)], sem1)
        ia.wait()
        ib.wait()

        pltpu.async_copy(g.at[rowb.at[0]], buf0, sem0)
        pltpu.async_copy(g.at[rowb.at[1]], buf1, sem1)

        def body(i, _):
            j0 = 2 * i
            pltpu.make_async_copy(g.at[rowb.at[j0]], buf0, sem0).wait()
            pltpu.sync_copy(buf0, accsp.at[colb.at[j0]], add=True)
            pltpu.async_copy(g.at[rowb.at[j0 + 2]], buf0, sem0)
            pltpu.make_async_copy(g.at[rowb.at[j0 + 1]], buf1, sem1).wait()
            pltpu.sync_copy(buf1, accsp.at[colb.at[j0 + 1]], add=True)
            pltpu.async_copy(g.at[rowb.at[j0 + 3]], buf1, sem1)
            return 0
        lax.fori_loop(0, nchunks // 2 - 1, body, 0)

        pltpu.make_async_copy(g.at[rowb.at[nchunks - 2]], buf0, sem0).wait()
        pltpu.sync_copy(buf0, accsp.at[colb.at[nchunks - 2]], add=True)
        pltpu.make_async_copy(g.at[rowb.at[nchunks - 1]], buf1, sem1).wait()
        pltpu.sync_copy(buf1, accsp.at[colb.at[nchunks - 1]], add=True)

    @pl.when(wid < NW - 1)
    def _():
        run_block(base, B0)
        run_block(base + B0, B0)

    @pl.when(wid == NW - 1)
    def _():
        run_block(base, LAST)

    plsc.subcore_barrier()
    pltpu.sync_copy(accsp.at[pl.ds(s * RPT, RPT)], out.at[c, pl.ds(s * RPT, RPT)])


def _sc_aggregate(g, er):
    return pl.kernel(
        _agg_body,
        out_type=jax.ShapeDtypeStruct((NC, NPA, D), jnp.float32),
        mesh=plsc.VectorSubcoreMesh(**_MESH),
        scratch_types=[
            pltpu.VMEM((B0, CH), jnp.int32),
            pltpu.VMEM((B0, CH), jnp.int32),
            pltpu.VMEM((CH, D), jnp.float32),
            pltpu.VMEM((CH, D), jnp.float32),
            pltpu.VMEM_SHARED((NPA, D), jnp.float32),
            pltpu.SemaphoreType.DMA,
            pltpu.SemaphoreType.DMA,
        ],
    )(g, er)


_R = 1000


def _dis_of(deg_ref):
    deg = 1.0 + jnp.sum(deg_ref[...], axis=1, keepdims=True)
    return lax.rsqrt(deg)


def _tcmm_body(x_ref, w_ref, h_ref):
    h_ref[...] = jnp.dot(x_ref[...], w_ref[...],
                         preferred_element_type=jnp.float32)


def _tc_mm(x, W1):
    return pl.pallas_call(
        _tcmm_body,
        grid=(N // _R,),
        in_specs=[
            pl.BlockSpec((_R, D), lambda i: (i, 0)),
            pl.BlockSpec((D, D), lambda i: (0, 0)),
        ],
        out_specs=pl.BlockSpec((_R, D), lambda i: (i, 0)),
        out_shape=jax.ShapeDtypeStruct((N, D), jnp.float32),
    )(x, W1)


def _tcscale_body(h_ref, deg_ref, g_ref):
    g_ref[...] = _dis_of(deg_ref) * h_ref[...]


def _tc_scale(h1, degp):
    return pl.pallas_call(
        _tcscale_body,
        grid=(N // _R,),
        in_specs=[
            pl.BlockSpec((_R, D), lambda i: (i, 0)),
            pl.BlockSpec((_R, NW), lambda i: (i, 0)),
        ],
        out_specs=pl.BlockSpec((_R, D), lambda i: (i, 0)),
        out_shape=jax.ShapeDtypeStruct((N, D), jnp.float32),
    )(h1, degp)


def _tc2_body(acc_ref, h_ref, deg_ref, b_ref, w_ref, h2_ref, g2_ref):
    a = acc_ref[0] + acc_ref[1]
    dis = _dis_of(deg_ref)
    pre = dis * a + dis * dis * h_ref[...] + b_ref[...]
    r = jnp.maximum(pre, 0.0)
    h2 = jnp.dot(r, w_ref[...], preferred_element_type=jnp.float32)
    h2_ref[...] = h2
    g2_ref[...] = dis * h2


def _tc_mid(acc1, h1, degp, b1, W2):
    return pl.pallas_call(
        _tc2_body,
        grid=(N // _R,),
        in_specs=[
            pl.BlockSpec((NC, _R, D), lambda i: (0, i, 0)),
            pl.BlockSpec((_R, D), lambda i: (i, 0)),
            pl.BlockSpec((_R, NW), lambda i: (i, 0)),
            pl.BlockSpec((1, D), lambda i: (0, 0)),
            pl.BlockSpec((D, D), lambda i: (0, 0)),
        ],
        out_specs=[
            pl.BlockSpec((_R, D), lambda i: (i, 0)),
            pl.BlockSpec((_R, D), lambda i: (i, 0)),
        ],
        out_shape=[
            jax.ShapeDtypeStruct((N, D), jnp.float32),
            jax.ShapeDtypeStruct((N, D), jnp.float32),
        ],
    )(acc1, h1, degp, b1, W2)


def _tc3_body(acc_ref, h_ref, deg_ref, b_ref, out_ref):
    a = acc_ref[0] + acc_ref[1]
    dis = _dis_of(deg_ref)
    out_ref[...] = dis * a + dis * dis * h_ref[...] + b_ref[...]


def _tc_final(acc2, h2, degp, b2):
    return pl.pallas_call(
        _tc3_body,
        grid=(N // _R,),
        in_specs=[
            pl.BlockSpec((NC, _R, D), lambda i: (0, i, 0)),
            pl.BlockSpec((_R, D), lambda i: (i, 0)),
            pl.BlockSpec((_R, NW), lambda i: (i, 0)),
            pl.BlockSpec((1, D), lambda i: (0, 0)),
        ],
        out_specs=pl.BlockSpec((_R, D), lambda i: (i, 0)),
        out_shape=jax.ShapeDtypeStruct((N, D), jnp.float32),
    )(acc2, h2, degp, b2)


def kernel(x, edge_index, W1, b1, W2, b2):
    k = jnp.arange(EP - E, dtype=jnp.int32)
    pad = jnp.stack([k % N, N + k % (NPA - N)])
    er = jnp.concatenate([edge_index, pad], axis=1).reshape(2, TCH, CH)
    b1r = b1.reshape(1, D)
    b2r = b2.reshape(1, D)

    h1 = _tc_mm(x, W1)
    degp = jnp.transpose(_sc_degree(er))
    g1 = _tc_scale(h1, degp)
    acc1 = _sc_aggregate(g1, er)
    h2, g2 = _tc_mid(acc1, h1, degp, b1r, W2)
    acc2 = _sc_aggregate(g2, er)
    return _tc_final(acc2, h2, degp, b2r)
